# baseline (speedup 1.0000x reference)
import jax
import jax.numpy as jnp
from jax.experimental import pallas as pl
from jax.experimental.pallas import tpu as pltpu

_K = 8192
_KB = 512
_T = 1024
_WIN = 8


def _vq_kernel(a_ref, zb_ref, e_ref, out_ref, wv_s, wi_s, gv_s, gi_s):
    k = pl.program_id(1)
    nk = pl.num_programs(1)
    zb = zb_ref[0].astype(jnp.float32)
    e = e_ref[...]
    conv = jax.lax.dot_general(
        e, zb, dimension_numbers=(((1,), (0,)), ((), ())),
        preferred_element_type=jnp.float32)
    e2 = jnp.sum(e * e, axis=1, keepdims=True)
    a = a_ref[0]
    dist = (a - conv) + e2
    m = jnp.min(dist, axis=0, keepdims=True)
    ids = jax.lax.broadcasted_iota(jnp.int32, (_KB, _T), 0) + k * _KB
    bidx = jnp.min(jnp.where(dist == m, ids, _K), axis=0, keepdims=True)

    @pl.when(k % _WIN == 0)
    def _():
        wv_s[...] = m
        wi_s[...] = bidx

    @pl.when(k % _WIN != 0)
    def _():
        better = m < wv_s[...]
        wi_s[...] = jnp.where(better, bidx, wi_s[...])
        wv_s[...] = jnp.where(better, m, wv_s[...])

    @pl.when(k % _WIN == _WIN - 1)
    def _():
        wv = wv_s[...]
        wi = wi_s[...]

        @pl.when(k == _WIN - 1)
        def _():
            gv_s[...] = wv.astype(jnp.bfloat16).astype(jnp.float32)
            gi_s[...] = wi

        @pl.when(k != _WIN - 1)
        def _():
            gv = gv_s[...]
            gi = gi_s[...]
            keep_v = gv < wv
            keep_i = keep_v | ((gv == wv) & (gi < wi))
            gv_s[...] = jnp.where(keep_v, gv, wv).astype(
                jnp.bfloat16).astype(jnp.float32)
            gi_s[...] = jnp.where(keep_i, gi, wi)

    @pl.when(k == nk - 1)
    def _():
        out_ref[...] = gi_s[...].reshape(1, 1, _T)


def kernel(z_e_x, embedding_weight):
    B, D, H, W = z_e_x.shape
    nt = H * W
    zb = (2.0 * z_e_x).astype(jnp.bfloat16).reshape(B, D, nt)
    a = jnp.sum(z_e_x * z_e_x, axis=1).reshape(B, 1, nt)

    grid = (B, _K // _KB)
    out = pl.pallas_call(
        _vq_kernel,
        grid=grid,
        in_specs=[
            pl.BlockSpec((1, 1, nt), lambda b, k: (b, 0, 0)),
            pl.BlockSpec((1, D, nt), lambda b, k: (b, 0, 0)),
            pl.BlockSpec((_KB, D), lambda b, k: (k, 0)),
        ],
        out_specs=pl.BlockSpec((1, 1, nt), lambda b, k: (b, 0, 0)),
        out_shape=jax.ShapeDtypeStruct((B, 1, nt), jnp.int32),
        scratch_shapes=[
            pltpu.VMEM((1, _T), jnp.float32),
            pltpu.VMEM((1, _T), jnp.int32),
            pltpu.VMEM((1, _T), jnp.float32),
            pltpu.VMEM((1, _T), jnp.int32),
        ],
        compiler_params=pltpu.CompilerParams(
            dimension_semantics=("parallel", "arbitrary")),
    )(a, zb, embedding_weight)
    return out.reshape(B, H, W)
